# baseline (speedup 1.0000x reference)
import functools

import jax
import jax.numpy as jnp
from jax import lax
from jax.experimental import pallas as pl
from jax.experimental.pallas import tpu as pltpu
from jax.experimental.pallas import tpu_sc as plsc

VOCAB = 32768
D = 64
LANES = 128
NS = 200
ST = NS // 8
BT = 32
DT = D // 8
SKEW = 136
SPS = 2
NBUF = 3
NSTEP = NS // SPS


def _make_kernel():
    mesh = plsc.VectorSubcoreMesh(core_axis_name="c", subcore_axis_name="s")

    @functools.partial(
        pl.kernel,
        mesh=mesh,
        out_type=jax.ShapeDtypeStruct((NS, DT, BT, 8, LANES), jnp.float32),
        scratch_types=[
            pltpu.VMEM((ST, 8 * LANES), jnp.int32),
            pltpu.VMEM((NBUF, SPS * LANES, D), jnp.float32),
            pltpu.VMEM((NBUF, SPS, DT, 8, SKEW), jnp.float32),
        ]
        + [pltpu.SemaphoreType.DMA] * NBUF
        + [pltpu.SemaphoreType.DMA] * NBUF,
        compiler_params=pltpu.CompilerParams(
            use_tc_tiling_on_sc=False, needs_layout_passes=False
        ),
    )
    def k(x_hbm, table_hbm, out_hbm, idx_v, g_v, t_v, *sems):
        gsem = sems[:NBUF]
        ssem = sems[NBUF:]
        bt = lax.axis_index("s") * 2 + lax.axis_index("c")

        pltpu.sync_copy(x_hbm.at[:, bt], idx_v)

        def start_gather(step, slot):
            s = step * SPS
            pltpu.make_async_copy(
                table_hbm.at[idx_v.at[s // 8, pl.ds((s % 8) * LANES, SPS * LANES)]],
                g_v.at[slot],
                gsem[slot],
            ).start()

        def wait_gather(slot):
            pltpu.make_async_copy(
                table_hbm.at[idx_v.at[0, pl.ds(0, SPS * LANES)]],
                g_v.at[slot],
                gsem[slot],
            ).wait()

        def start_store(step, slot):
            pltpu.make_async_copy(
                t_v.at[slot, :, :, :, pl.ds(0, LANES)],
                out_hbm.at[pl.ds(step * SPS, SPS), :, bt],
                ssem[slot],
            ).start()

        def wait_store(slot):
            pltpu.make_async_copy(
                t_v.at[0, :, :, :, pl.ds(0, LANES)],
                out_hbm.at[pl.ds(0, SPS), :, bt],
                ssem[slot],
            ).wait()

        def transpose(slot, h):
            g = g_v.at[slot, pl.ds(h * LANES, LANES)]
            t = t_v.at[slot, h]
            iota = lax.iota(jnp.int32, 16)
            dt_idx = [(d0 + iota) // 8 for d0 in range(0, D, 16)]
            dr_idx = [(d0 + iota) % 8 for d0 in range(0, D, 16)]

            @plsc.parallel_loop(0, LANES)
            def b_body(b):
                bvec = jnp.full((16,), 0, jnp.int32) + b
                for j, d0 in enumerate(range(0, D, 16)):
                    vec = g[b, pl.ds(d0, 16)]
                    plsc.store_scatter(t, [dt_idx[j], dr_idx[j], bvec], vec)

        for p in range(NBUF - 1):
            start_gather(p, p)

        def body(i, _):
            for b in range(NBUF):
                step = i * NBUF + b
                slot = b

                @pl.when(step + NBUF - 1 < NSTEP)
                def _():
                    start_gather(step + NBUF - 1, (b + NBUF - 1) % NBUF)

                wait_gather(slot)

                @pl.when(step >= NBUF)
                def _():
                    wait_store(slot)

                for h in range(SPS):
                    transpose(slot, h)
                start_store(step, slot)
            return ()

        lax.fori_loop(0, NSTEP // NBUF, body, (), unroll=False)

        for step in range(NSTEP - NSTEP % NBUF, NSTEP):
            slot = step % NBUF
            wait_gather(slot)
            wait_store(slot)
            for h in range(SPS):
                transpose(slot, h)
            start_store(step, slot)

        for p in range(NBUF):
            wait_store(p)

    return k


def kernel(x, table):
    nb, ns = x.shape
    x4 = (
        x.astype(jnp.int32)
        .T.reshape(ST, 8, BT, LANES)
        .transpose(0, 2, 1, 3)
        .reshape(ST, BT, 8 * LANES)
    )
    out5 = _make_kernel()(x4, table)
    return out5.transpose(2, 4, 0, 1, 3).reshape(nb, ns, D)


# speedup vs baseline: 1.0062x; 1.0062x over previous
import functools

import numpy as np

import jax
import jax.numpy as jnp
from jax import lax
from jax.experimental import pallas as pl
from jax.experimental.pallas import tpu as pltpu
from jax.experimental.pallas import tpu_sc as plsc

VOCAB = 32768
D = 64
LANES = 128
NS = 200
ST = NS // 8
BT = 32
DT = D // 8
SKEW = 136
NBUF = 5


def _make_kernel():
    mesh = plsc.VectorSubcoreMesh(core_axis_name="c", subcore_axis_name="s")

    @functools.partial(
        pl.kernel,
        mesh=mesh,
        out_type=jax.ShapeDtypeStruct((NS, DT, BT, 8, LANES), jnp.float32),
        scratch_types=[
            pltpu.VMEM((ST, 8, LANES), jnp.int32),
            pltpu.VMEM((NBUF, LANES, D), jnp.float32),
            pltpu.VMEM((NBUF, DT, 8, SKEW), jnp.float32),
        ]
        + [pltpu.SemaphoreType.DMA] * NBUF
        + [pltpu.SemaphoreType.DMA] * NBUF,
        compiler_params=pltpu.CompilerParams(
            use_tc_tiling_on_sc=False, needs_layout_passes=False
        ),
    )
    def k(x_hbm, table_hbm, out_hbm, idx_v, g_v, t_v, *sems):
        gsem = sems[:NBUF]
        ssem = sems[NBUF:]
        bt = lax.axis_index("s") * 2 + lax.axis_index("c")

        pltpu.sync_copy(x_hbm.at[:, bt], idx_v)

        def start_gather(s, slot):
            pltpu.make_async_copy(
                table_hbm.at[idx_v.at[s // 8, s % 8]], g_v.at[slot], gsem[slot]
            ).start()

        def wait_gather(slot):
            pltpu.make_async_copy(
                table_hbm.at[idx_v.at[0, 0]], g_v.at[slot], gsem[slot]
            ).wait()

        def start_store(s, slot):
            pltpu.make_async_copy(
                t_v.at[slot, :, :, pl.ds(0, LANES)], out_hbm.at[s, :, bt], ssem[slot]
            ).start()

        def wait_store(slot):
            pltpu.make_async_copy(
                t_v.at[0, :, :, pl.ds(0, LANES)], out_hbm.at[0, :, bt], ssem[slot]
            ).wait()

        def transpose(slot):
            g = g_v.at[slot]
            t = t_v.at[slot]
            iota = lax.iota(jnp.int32, 16)
            dt_idx = [(d0 + iota) // 8 for d0 in range(0, D, 16)]
            dr_idx = [(d0 + iota) % 8 for d0 in range(0, D, 16)]

            @plsc.parallel_loop(0, LANES)
            def b_body(b):
                bvec = jnp.full((16,), 0, jnp.int32) + b
                for j, d0 in enumerate(range(0, D, 16)):
                    vec = g[b, pl.ds(d0, 16)]
                    plsc.store_scatter(t, [dt_idx[j], dr_idx[j], bvec], vec)

        for p in range(NBUF - 1):
            start_gather(p, p)

        def body(i, _):
            for b in range(NBUF):
                s = i * NBUF + b
                slot = b

                @pl.when(s + NBUF - 1 < NS)
                def _():
                    start_gather(s + NBUF - 1, (b + NBUF - 1) % NBUF)

                wait_gather(slot)

                @pl.when(s >= NBUF)
                def _():
                    wait_store(slot)

                transpose(slot)
                start_store(s, slot)
            return ()

        lax.fori_loop(0, NS // NBUF, body, (), unroll=False)
        for p in range(NBUF):
            wait_store(p)

    return k


def kernel(x, table):
    nb, ns = x.shape
    x4 = (
        x.astype(jnp.int32)
        .T.reshape(ST, 8, BT, LANES)
        .transpose(0, 2, 1, 3)
    )
    out5 = _make_kernel()(x4, table)
    return out5.transpose(2, 4, 0, 1, 3).reshape(nb, ns, D)
